# baseline (speedup 1.0000x reference)
import functools

import jax
import jax.numpy as jnp
from jax import lax
from jax.experimental import pallas as pl
from jax.experimental.pallas import tpu as pltpu
from jax.experimental.pallas import tpu_sc as plsc

_NC = 2
_NS = 16
_NW = _NC * _NS
_CH = 128


def _node_proj(nfeat, w1_src, w1_dst):
    n, d = nfeat.shape
    h = w1_src.shape[1]

    def body(n_ref, ws_ref, wd_ref, t_ref):
        x = n_ref[...]
        p = jnp.dot(x, ws_ref[...], preferred_element_type=jnp.float32)
        q = jnp.dot(x, wd_ref[...], preferred_element_type=jnp.float32)
        pb = lax.bitcast_convert_type(
            p.astype(jnp.bfloat16).astype(jnp.float32), jnp.int32)
        qb = lax.bitcast_convert_type(
            q.astype(jnp.bfloat16).astype(jnp.float32), jnp.int32)
        t_ref[...] = jnp.bitwise_or(
            lax.shift_right_logical(pb, 16),
            jnp.bitwise_and(qb, jnp.int32(-65536)))

    return pl.pallas_call(
        body,
        out_shape=jax.ShapeDtypeStruct((n, h), jnp.int32),
    )(nfeat, w1_src, w1_dst)


def _sc_gather(table, src_pad, dst_pad, seg_off, half_rows, cph):
    n, d = table.shape
    mesh = plsc.VectorSubcoreMesh(
        core_axis_name="c", subcore_axis_name="s", num_cores=1)
    nbuf = 2
    pf = 1
    cpw = 2 * cph
    slab = cph * _CH

    @functools.partial(
        pl.kernel,
        mesh=mesh,
        out_type=(jax.ShapeDtypeStruct((half_rows, d), jnp.int32),
                  jax.ShapeDtypeStruct((half_rows, d), jnp.int32)),
        scratch_types=[
            pltpu.VMEM((cpw * _CH,), jnp.int32),
            pltpu.VMEM_SHARED((n, d), jnp.int32),
        ]
        + [pltpu.VMEM((_CH, d), jnp.int32)] * nbuf
        + [pltpu.SemaphoreType.DMA] * (2 * nbuf + 1),
    )
    def gather_k(t_hbm, src_hbm, dst_hbm, outs_hbm, outd_hbm, idx_v, t_sh,
                 *bufs_and_sems):
        rbufs = bufs_and_sems[:nbuf]
        gsems = bufs_and_sems[nbuf:2 * nbuf]
        wsems = bufs_and_sems[2 * nbuf:3 * nbuf]
        tsem = bufs_and_sems[3 * nbuf]
        s = lax.axis_index("s")
        base = s * slab

        @pl.when(s == 0)
        def _():
            pltpu.async_copy(t_hbm, t_sh, tsem).wait()

        pltpu.sync_copy(src_hbm.at[pl.ds(seg_off + base, slab)],
                        idx_v.at[pl.ds(0, slab)])
        pltpu.sync_copy(dst_hbm.at[pl.ds(seg_off + base, slab)],
                        idx_v.at[pl.ds(slab, slab)])
        plsc.subcore_barrier()

        def gat(jj, b):
            return pltpu.make_async_copy(
                t_sh.at[idx_v.at[pl.ds(jj * _CH, _CH)]], rbufs[b], gsems[b])

        def wrb_start(jj, b):
            @pl.when(jj < cph)
            def _():
                pltpu.make_async_copy(
                    rbufs[b], outs_hbm.at[pl.ds(base + jj * _CH, _CH)],
                    wsems[b]).start()

            @pl.when(jj >= cph)
            def _():
                pltpu.make_async_copy(
                    rbufs[b],
                    outd_hbm.at[pl.ds(base + (jj - cph) * _CH, _CH)],
                    wsems[b]).start()

        def wrb_wait(b):
            pltpu.make_async_copy(
                rbufs[b], outs_hbm.at[pl.ds(base, _CH)], wsems[b]).wait()

        for b in range(pf):
            gat(b, b).start()

        @pl.loop(0, cpw, step=nbuf)
        def _(j):
            for b in range(nbuf):
                jj = j + b
                gat(jj, b).wait()
                wrb_start(jj, b)
                jn = jj + pf
                bn = (b + pf) % nbuf

                @pl.when(jn < cpw)
                def _():
                    @pl.when(jj >= nbuf - pf)
                    def _():
                        wrb_wait(bn)

                    gat(jn, bn).start()

        for b in range(nbuf):
            wrb_wait(b)

    return gather_k(table, src_pad, dst_pad)


def _mlp_core(x_ref, rs_ref, rd_ref, we_ref, w2_ref, b1_ref, b2_ref,
              g_ref, bb_ref, o_ref):
    x = x_ref[...]
    ps = lax.bitcast_convert_type(
        lax.shift_left(rs_ref[...], 16), jnp.float32)
    qd = lax.bitcast_convert_type(
        jnp.bitwise_and(rd_ref[...], jnp.int32(-65536)), jnp.float32)
    h = jnp.dot(x.astype(jnp.bfloat16), we_ref[...],
                preferred_element_type=jnp.float32)
    h = h + ps + qd + b1_ref[...]
    h = h * jax.nn.sigmoid(h)
    h2 = jnp.dot(h.astype(jnp.bfloat16), w2_ref[...],
                 preferred_element_type=jnp.float32)
    h2 = h2 + b2_ref[...]
    mu = jnp.mean(h2, axis=-1, keepdims=True)
    var = jnp.mean((h2 - mu) * (h2 - mu), axis=-1, keepdims=True)
    o_ref[...] = (h2 - mu) * lax.rsqrt(var + 1e-5) * g_ref[...] + bb_ref[...] + x


def _edge_mlp_seg(efeat, rows_s, rows_d, w1_edge, w2, b1, b2, ln_g, ln_b,
                  block, seg, nseg, buf):
    e, d = efeat.shape
    e_seg = e // nseg
    nblk = e_seg // block
    base_blk = seg * nblk
    hid = w1_edge.shape[1]
    out_dim = w2.shape[1]
    full = lambda *s: pl.BlockSpec(s, lambda i: tuple(0 for _ in s))
    in_specs = [
        pl.BlockSpec((block, d), lambda i: (base_blk + i, 0)),
        pl.BlockSpec((block, hid), lambda i: (i, 0)),
        pl.BlockSpec((block, hid), lambda i: (i, 0)),
        full(d, hid),
        full(hid, out_dim),
        full(1, hid),
        full(1, out_dim),
        full(1, out_dim),
        full(1, out_dim),
    ]
    operands = (efeat, rows_s, rows_d, w1_edge, w2, b1, b2, ln_g, ln_b)
    kwargs = {}
    body = _mlp_core
    if buf is not None:
        def body(x, rs, rd, we, w2r, b1r, b2r, gr, bbr, _buf, o):
            _mlp_core(x, rs, rd, we, w2r, b1r, b2r, gr, bbr, o)
        in_specs = in_specs + [pl.BlockSpec(memory_space=pl.ANY)]
        operands = operands + (buf,)
        kwargs = dict(input_output_aliases={9: 0})
    return pl.pallas_call(
        body,
        grid=(nblk,),
        in_specs=in_specs,
        out_specs=pl.BlockSpec((block, out_dim), lambda i: (base_blk + i, 0)),
        out_shape=jax.ShapeDtypeStruct((e, out_dim), jnp.float32),
        compiler_params=pltpu.CompilerParams(
            dimension_semantics=("arbitrary",),
        ),
        **kwargs,
    )(*operands)


def kernel(efeat, nfeat, edge_index, W1, b1, W2, b2, ln_g, ln_b):
    e, d_edge = efeat.shape
    n, d_node = nfeat.shape
    src = edge_index[0]
    dst = edge_index[1]

    table = _node_proj(nfeat, W1[d_edge:d_edge + d_node], W1[d_edge + d_node:])

    nseg = 8
    e_seg = e // nseg
    quantum = _NS * _CH
    half_rows = ((e_seg + quantum - 1) // quantum) * quantum
    cph = half_rows // _CH // _NS
    pad = half_rows - e_seg
    zpad = jnp.zeros((2, pad), dtype=jnp.int32)
    src_pad, dst_pad = jnp.concatenate([edge_index, zpad], axis=1)
    b1r, b2r = b1.reshape(1, -1), b2.reshape(1, -1)
    gr, br = ln_g.reshape(1, -1), ln_b.reshape(1, -1)
    w1e = W1[:d_edge].astype(jnp.bfloat16)
    w2b = W2.astype(jnp.bfloat16)

    buf = None
    for k in range(nseg):
        rs_k, rd_k = _sc_gather(table, src_pad, dst_pad, k * e_seg,
                                half_rows, cph)
        buf = _edge_mlp_seg(efeat, rs_k, rd_k, w1e, w2b, b1r, b2r, gr, br,
                            block=4000, seg=k, nseg=nseg, buf=buf)
    return (buf, nfeat)


# speedup vs baseline: 1.0775x; 1.0775x over previous
import functools

import jax
import jax.numpy as jnp
from jax import lax
from jax.experimental import pallas as pl
from jax.experimental.pallas import tpu as pltpu
from jax.experimental.pallas import tpu_sc as plsc

_NC = 2
_NS = 16
_NW = _NC * _NS
_CH = 128


def _node_proj(nfeat, w1_src, w1_dst):
    n, d = nfeat.shape
    h = w1_src.shape[1]

    def body(n_ref, ws_ref, wd_ref, t_ref):
        x = n_ref[...]
        p = jnp.dot(x, ws_ref[...], preferred_element_type=jnp.float32)
        q = jnp.dot(x, wd_ref[...], preferred_element_type=jnp.float32)
        pb = lax.bitcast_convert_type(
            p.astype(jnp.bfloat16).astype(jnp.float32), jnp.int32)
        qb = lax.bitcast_convert_type(
            q.astype(jnp.bfloat16).astype(jnp.float32), jnp.int32)
        t_ref[...] = jnp.bitwise_or(
            lax.shift_right_logical(pb, 16),
            jnp.bitwise_and(qb, jnp.int32(-65536)))

    return pl.pallas_call(
        body,
        out_shape=jax.ShapeDtypeStruct((n, h), jnp.int32),
    )(nfeat, w1_src, w1_dst)


def _sc_gather(table, src_pad, dst_pad, seg_off, half_rows, cph):
    n, d = table.shape
    mesh = plsc.VectorSubcoreMesh(
        core_axis_name="c", subcore_axis_name="s", num_cores=1)
    nbuf = 2
    pf = 1
    cpw = 2 * cph
    slab = cph * _CH

    @functools.partial(
        pl.kernel,
        mesh=mesh,
        out_type=(jax.ShapeDtypeStruct((half_rows, d), jnp.int32),
                  jax.ShapeDtypeStruct((half_rows, d), jnp.int32)),
        scratch_types=[
            pltpu.VMEM((cpw * _CH,), jnp.int32),
            pltpu.VMEM_SHARED((n, d), jnp.int32),
        ]
        + [pltpu.VMEM((_CH, d), jnp.int32)] * nbuf
        + [pltpu.SemaphoreType.DMA] * (2 * nbuf + 1),
    )
    def gather_k(t_hbm, src_hbm, dst_hbm, outs_hbm, outd_hbm, idx_v, t_sh,
                 *bufs_and_sems):
        rbufs = bufs_and_sems[:nbuf]
        gsems = bufs_and_sems[nbuf:2 * nbuf]
        wsems = bufs_and_sems[2 * nbuf:3 * nbuf]
        tsem = bufs_and_sems[3 * nbuf]
        s = lax.axis_index("s")
        base = s * slab

        @pl.when(s == 0)
        def _():
            pltpu.async_copy(t_hbm, t_sh, tsem).wait()

        pltpu.sync_copy(src_hbm.at[pl.ds(seg_off + base, slab)],
                        idx_v.at[pl.ds(0, slab)])
        pltpu.sync_copy(dst_hbm.at[pl.ds(seg_off + base, slab)],
                        idx_v.at[pl.ds(slab, slab)])
        plsc.subcore_barrier()

        def gat(jj, b):
            return pltpu.make_async_copy(
                t_sh.at[idx_v.at[pl.ds(jj * _CH, _CH)]], rbufs[b], gsems[b])

        def wrb_start(jj, b):
            @pl.when(jj < cph)
            def _():
                pltpu.make_async_copy(
                    rbufs[b], outs_hbm.at[pl.ds(base + jj * _CH, _CH)],
                    wsems[b]).start()

            @pl.when(jj >= cph)
            def _():
                pltpu.make_async_copy(
                    rbufs[b],
                    outd_hbm.at[pl.ds(base + (jj - cph) * _CH, _CH)],
                    wsems[b]).start()

        def wrb_wait(b):
            pltpu.make_async_copy(
                rbufs[b], outs_hbm.at[pl.ds(base, _CH)], wsems[b]).wait()

        for b in range(pf):
            gat(b, b).start()

        @pl.loop(0, cpw, step=nbuf)
        def _(j):
            for b in range(nbuf):
                jj = j + b
                gat(jj, b).wait()
                wrb_start(jj, b)
                jn = jj + pf
                bn = (b + pf) % nbuf

                @pl.when(jn < cpw)
                def _():
                    @pl.when(jj >= nbuf - pf)
                    def _():
                        wrb_wait(bn)

                    gat(jn, bn).start()

        for b in range(nbuf):
            wrb_wait(b)

    return gather_k(table, src_pad, dst_pad)


def _mlp_core(x_ref, rs_ref, rd_ref, we_ref, w2_ref, b1_ref, b2_ref,
              g_ref, bb_ref, o_ref):
    x = x_ref[...]
    ps = lax.bitcast_convert_type(
        lax.shift_left(rs_ref[...], 16), jnp.float32)
    qd = lax.bitcast_convert_type(
        jnp.bitwise_and(rd_ref[...], jnp.int32(-65536)), jnp.float32)
    h = jnp.dot(x.astype(jnp.bfloat16), we_ref[...],
                preferred_element_type=jnp.float32)
    h = h + ps + qd + b1_ref[...]
    h = h * jax.nn.sigmoid(h)
    h2 = jnp.dot(h.astype(jnp.bfloat16), w2_ref[...],
                 preferred_element_type=jnp.float32)
    h2 = h2 + b2_ref[...]
    mu = jnp.mean(h2, axis=-1, keepdims=True)
    var = jnp.mean((h2 - mu) * (h2 - mu), axis=-1, keepdims=True)
    o_ref[...] = (h2 - mu) * lax.rsqrt(var + 1e-5) * g_ref[...] + bb_ref[...] + x


def _edge_mlp_seg(efeat, rows_s, rows_d, w1_edge, w2, b1, b2, ln_g, ln_b,
                  block, seg, nseg, buf):
    e, d = efeat.shape
    e_seg = e // nseg
    nblk = e_seg // block
    base_blk = seg * nblk
    hid = w1_edge.shape[1]
    out_dim = w2.shape[1]
    full = lambda *s: pl.BlockSpec(s, lambda i: tuple(0 for _ in s))
    in_specs = [
        pl.BlockSpec((block, d), lambda i: (base_blk + i, 0)),
        pl.BlockSpec((block, hid), lambda i: (i, 0)),
        pl.BlockSpec((block, hid), lambda i: (i, 0)),
        full(d, hid),
        full(hid, out_dim),
        full(1, hid),
        full(1, out_dim),
        full(1, out_dim),
        full(1, out_dim),
    ]
    operands = (efeat, rows_s, rows_d, w1_edge, w2, b1, b2, ln_g, ln_b)
    kwargs = {}
    body = _mlp_core
    if buf is not None:
        def body(x, rs, rd, we, w2r, b1r, b2r, gr, bbr, _buf, o):
            _mlp_core(x, rs, rd, we, w2r, b1r, b2r, gr, bbr, o)
        in_specs = in_specs + [pl.BlockSpec(memory_space=pl.ANY)]
        operands = operands + (buf,)
        kwargs = dict(input_output_aliases={9: 0})
    return pl.pallas_call(
        body,
        grid=(nblk,),
        in_specs=in_specs,
        out_specs=pl.BlockSpec((block, out_dim), lambda i: (base_blk + i, 0)),
        out_shape=jax.ShapeDtypeStruct((e, out_dim), jnp.float32),
        compiler_params=pltpu.CompilerParams(
            dimension_semantics=("arbitrary",),
        ),
        **kwargs,
    )(*operands)


def kernel(efeat, nfeat, edge_index, W1, b1, W2, b2, ln_g, ln_b):
    e, d_edge = efeat.shape
    n, d_node = nfeat.shape
    src = edge_index[0]
    dst = edge_index[1]

    table = _node_proj(nfeat, W1[d_edge:d_edge + d_node], W1[d_edge + d_node:])

    nseg = 5
    e_seg = e // nseg
    quantum = _NS * _CH
    half_rows = ((e_seg + quantum - 1) // quantum) * quantum
    cph = half_rows // _CH // _NS
    pad = half_rows - e_seg
    zpad = jnp.zeros((2, pad), dtype=jnp.int32)
    src_pad, dst_pad = jnp.concatenate([edge_index, zpad], axis=1)
    b1r, b2r = b1.reshape(1, -1), b2.reshape(1, -1)
    gr, br = ln_g.reshape(1, -1), ln_b.reshape(1, -1)
    w1e = W1[:d_edge].astype(jnp.bfloat16)
    w2b = W2.astype(jnp.bfloat16)

    buf = None
    for k in range(nseg):
        rs_k, rd_k = _sc_gather(table, src_pad, dst_pad, k * e_seg,
                                half_rows, cph)
        buf = _edge_mlp_seg(efeat, rs_k, rd_k, w1e, w2b, b1r, b2r, gr, br,
                            block=4000, seg=k, nseg=nseg, buf=buf)
    return (buf, nfeat)


# speedup vs baseline: 1.0790x; 1.0014x over previous
import functools

import jax
import jax.numpy as jnp
from jax import lax
from jax.experimental import pallas as pl
from jax.experimental.pallas import tpu as pltpu
from jax.experimental.pallas import tpu_sc as plsc

_NS = 16
_CH = 128


def _node_proj(nfeat, w1_src, w1_dst):
    n, d = nfeat.shape
    h = w1_src.shape[1]

    def body(n_ref, ws_ref, wd_ref, t_ref):
        x = n_ref[...]
        p = jnp.dot(x, ws_ref[...], preferred_element_type=jnp.float32)
        q = jnp.dot(x, wd_ref[...], preferred_element_type=jnp.float32)
        pb = lax.bitcast_convert_type(
            p.astype(jnp.bfloat16).astype(jnp.float32), jnp.int32)
        qb = lax.bitcast_convert_type(
            q.astype(jnp.bfloat16).astype(jnp.float32), jnp.int32)
        t_ref[...] = jnp.bitwise_or(
            lax.shift_right_logical(pb, 16),
            jnp.bitwise_and(qb, jnp.int32(-65536)))

    return pl.pallas_call(
        body,
        out_shape=jax.ShapeDtypeStruct((n, h), jnp.int32),
    )(nfeat, w1_src, w1_dst)


def _sc_gather(table, src_pad, dst_pad, seg_off, half_rows, cph):
    n, d = table.shape
    mesh = plsc.VectorSubcoreMesh(
        core_axis_name="c", subcore_axis_name="s", num_cores=1)
    nbuf = 2
    pf = 1
    cpw = 2 * cph
    slab = cph * _CH

    @functools.partial(
        pl.kernel,
        mesh=mesh,
        out_type=(jax.ShapeDtypeStruct((half_rows, d), jnp.int32),
                  jax.ShapeDtypeStruct((half_rows, d), jnp.int32)),
        scratch_types=[
            pltpu.VMEM((cpw * _CH,), jnp.int32),
            pltpu.VMEM_SHARED((n, d), jnp.int32),
        ]
        + [pltpu.VMEM((_CH, d), jnp.int32)] * nbuf
        + [pltpu.SemaphoreType.DMA] * (2 * nbuf + 1),
    )
    def gather_k(t_hbm, src_hbm, dst_hbm, outs_hbm, outd_hbm, idx_v, t_sh,
                 *bufs_and_sems):
        rbufs = bufs_and_sems[:nbuf]
        gsems = bufs_and_sems[nbuf:2 * nbuf]
        wsems = bufs_and_sems[2 * nbuf:3 * nbuf]
        tsem = bufs_and_sems[3 * nbuf]
        s = lax.axis_index("s")
        base = s * slab

        @pl.when(s == 0)
        def _():
            pltpu.async_copy(t_hbm, t_sh, tsem).wait()

        pltpu.sync_copy(src_hbm.at[pl.ds(seg_off + base, slab)],
                        idx_v.at[pl.ds(0, slab)])
        pltpu.sync_copy(dst_hbm.at[pl.ds(seg_off + base, slab)],
                        idx_v.at[pl.ds(slab, slab)])
        plsc.subcore_barrier()

        def gat(jj, b):
            return pltpu.make_async_copy(
                t_sh.at[idx_v.at[pl.ds(jj * _CH, _CH)]], rbufs[b], gsems[b])

        def wrb_start(jj, b):
            @pl.when(jj < cph)
            def _():
                pltpu.make_async_copy(
                    rbufs[b], outs_hbm.at[pl.ds(base + jj * _CH, _CH)],
                    wsems[b]).start()

            @pl.when(jj >= cph)
            def _():
                pltpu.make_async_copy(
                    rbufs[b],
                    outd_hbm.at[pl.ds(base + (jj - cph) * _CH, _CH)],
                    wsems[b]).start()

        def wrb_wait(b):
            pltpu.make_async_copy(
                rbufs[b], outs_hbm.at[pl.ds(base, _CH)], wsems[b]).wait()

        for b in range(pf):
            gat(b, b).start()

        @pl.loop(0, cpw, step=nbuf)
        def _(j):
            for b in range(nbuf):
                jj = j + b
                gat(jj, b).wait()
                wrb_start(jj, b)
                jn = jj + pf
                bn = (b + pf) % nbuf

                @pl.when(jn < cpw)
                def _():
                    @pl.when(jj >= nbuf - pf)
                    def _():
                        wrb_wait(bn)

                    gat(jn, bn).start()

        for b in range(nbuf):
            wrb_wait(b)

    return gather_k(table, src_pad, dst_pad)


def _mlp_core(x_ref, rs_ref, rd_ref, we_ref, w2_ref, b1_ref, b2_ref,
              g_ref, bb_ref, o_ref):
    x = x_ref[...]
    ps = lax.bitcast_convert_type(
        lax.shift_left(rs_ref[...], 16), jnp.float32)
    qd = lax.bitcast_convert_type(
        jnp.bitwise_and(rd_ref[...], jnp.int32(-65536)), jnp.float32)
    h = jnp.dot(x.astype(jnp.bfloat16), we_ref[...],
                preferred_element_type=jnp.float32)
    h = h + ps + qd + b1_ref[...]
    h = h * jax.nn.sigmoid(h)
    h2 = jnp.dot(h.astype(jnp.bfloat16), w2_ref[...],
                 preferred_element_type=jnp.float32)
    h2 = h2 + b2_ref[...]
    mu = jnp.mean(h2, axis=-1, keepdims=True)
    var = jnp.mean((h2 - mu) * (h2 - mu), axis=-1, keepdims=True)
    o_ref[...] = (h2 - mu) * lax.rsqrt(var + 1e-5) * g_ref[...] + bb_ref[...] + x


def _edge_mlp_seg(efeat, rows_s, rows_d, w1_edge, w2, b1, b2, ln_g, ln_b,
                  block, seg, nseg, buf):
    e, d = efeat.shape
    e_seg = e // nseg
    nblk = e_seg // block
    base_blk = seg * nblk
    hid = w1_edge.shape[1]
    out_dim = w2.shape[1]
    full = lambda *s: pl.BlockSpec(s, lambda i: tuple(0 for _ in s))
    in_specs = [
        pl.BlockSpec((block, d), lambda i: (base_blk + i, 0)),
        pl.BlockSpec((block, hid), lambda i: (i, 0)),
        pl.BlockSpec((block, hid), lambda i: (i, 0)),
        full(d, hid),
        full(hid, out_dim),
        full(1, hid),
        full(1, out_dim),
        full(1, out_dim),
        full(1, out_dim),
    ]
    operands = (efeat, rows_s, rows_d, w1_edge, w2, b1, b2, ln_g, ln_b)
    kwargs = {}
    body = _mlp_core
    if buf is not None:
        def body(x, rs, rd, we, w2r, b1r, b2r, gr, bbr, _buf, o):
            _mlp_core(x, rs, rd, we, w2r, b1r, b2r, gr, bbr, o)
        in_specs = in_specs + [pl.BlockSpec(memory_space=pl.ANY)]
        operands = operands + (buf,)
        kwargs = dict(input_output_aliases={9: 0})
    return pl.pallas_call(
        body,
        grid=(nblk,),
        in_specs=in_specs,
        out_specs=pl.BlockSpec((block, out_dim), lambda i: (base_blk + i, 0)),
        out_shape=jax.ShapeDtypeStruct((e, out_dim), jnp.float32),
        compiler_params=pltpu.CompilerParams(
            dimension_semantics=("arbitrary",),
        ),
        **kwargs,
    )(*operands)


def kernel(efeat, nfeat, edge_index, W1, b1, W2, b2, ln_g, ln_b):
    e, d_edge = efeat.shape
    n, d_node = nfeat.shape

    table = _node_proj(nfeat, W1[d_edge:d_edge + d_node], W1[d_edge + d_node:])

    nseg = 5
    e_seg = e // nseg
    quantum = _NS * _CH
    half_rows = ((e_seg + quantum - 1) // quantum) * quantum
    cph = half_rows // _CH // _NS
    pad = half_rows - e_seg
    zpad = jnp.zeros((2, pad), dtype=jnp.int32)
    src_pad, dst_pad = jnp.concatenate([edge_index, zpad], axis=1)
    b1r, b2r = b1.reshape(1, -1), b2.reshape(1, -1)
    gr, br = ln_g.reshape(1, -1), ln_b.reshape(1, -1)
    w1e = W1[:d_edge].astype(jnp.bfloat16)
    w2b = W2.astype(jnp.bfloat16)

    buf = None
    for k in range(nseg):
        rs_k, rd_k = _sc_gather(table, src_pad, dst_pad, k * e_seg,
                                half_rows, cph)
        buf = _edge_mlp_seg(efeat, rs_k, rd_k, w1e, w2b, b1r, b2r, gr, br,
                            block=4000, seg=k, nseg=nseg, buf=buf)
    return (buf, nfeat)
